# baseline (speedup 1.0000x reference)
import functools

import jax
import jax.numpy as jnp
from jax import lax
from jax.experimental import pallas as pl
from jax.experimental.pallas import tpu as pltpu
from jax.experimental.pallas import tpu_sc as plsc

_K = 16
_GRP = 16
_FOLDS = [(1, 64), (1, 32), (1, 16), (1, 8), (1, 4), (1, 2), (1, 1),
          (0, 4), (0, 2), (0, 1)]


_NC = 2
_NS = 16
_NW = _NC * _NS
_CH = 49152
_UNROLL = 4


def _sc_merge_equal(a, b):
    d = len(a)
    v = list(a) + list(b)[::-1]
    dist = d
    while dist >= 1:
        for i in range(2 * d):
            if not i & dist:
                hi = jnp.maximum(v[i], v[i + dist])
                lo = jnp.minimum(v[i], v[i + dist])
                v[i], v[i + dist] = hi, lo
        dist //= 2
    return v


def _reduce16(vec, redscr, op):
    del redscr
    vals = [vec[i] for i in range(16)]
    while len(vals) > 1:
        vals = [op(vals[i], vals[i + 1]) for i in range(0, len(vals), 2)]
    return vals[0]


def _sc_16th_largest(cands, redscr):
    one = jnp.full((16,), 1.0, jnp.float32)
    zero16 = jnp.zeros((16,), jnp.float32)
    neginf16 = jnp.full((16,), -jnp.inf, jnp.float32)
    tau = jnp.float32(jnp.inf)
    acc = jnp.float32(0.0)
    t0 = jnp.float32(0.0)
    for _ in range(_K):
        masked = [jnp.where(v < tau, v, neginf16) for v in cands]
        mm = masked[0]
        for v in masked[1:]:
            mm = jnp.maximum(mm, v)
        m = _reduce16(mm, redscr, jnp.maximum)
        cntv = zero16
        for v in cands:
            cntv = cntv + jnp.where(v == m, one, zero16)
        cnt = _reduce16(cntv, redscr, lambda a, b: a + b)
        t0 = jnp.where(acc < jnp.float32(_K), m, t0)
        acc = acc + cnt
        tau = m
    return t0


def _sc_row_body(x_hbm, buf, redscr, row, hw):
    nchunks = hw // _CH
    nb = _UNROLL
    neg = jnp.full((16,), -jnp.inf, jnp.float32)

    m1 = (neg,) * nb
    m2 = (neg,) * nb
    m3 = (neg,) * nb
    for c in range(nchunks):
        pltpu.sync_copy(x_hbm.at[pl.ds(row * hw + c * _CH, _CH)], buf)

        def p1_body(i, carry):
            c1, c2, c3 = list(carry[0]), list(carry[1]), list(carry[2])
            for u in range(nb):
                v = buf[pl.ds((i * nb + u) * 16, 16)]
                t = jnp.minimum(c1[u], v)
                c1[u] = jnp.maximum(c1[u], v)
                q = jnp.minimum(c2[u], t)
                c2[u] = jnp.maximum(c2[u], t)
                c3[u] = jnp.maximum(c3[u], q)
            return tuple(c1), tuple(c2), tuple(c3)

        m1, m2, m3 = lax.fori_loop(0, _CH // (16 * nb), p1_body,
                                   (m1, m2, m3))

    lists = [(m1[p], m2[p], m3[p], neg) for p in range(nb)]
    while len(lists) > 1:
        nxt = []
        for i in range(0, len(lists), 2):
            nxt.append(tuple(_sc_merge_equal(lists[i], lists[i + 1])[:4]))
        lists = nxt
    t0 = _sc_16th_largest(list(lists[0]), redscr)
    t0v = jnp.full((16,), t0, jnp.float32)

    zero16 = jnp.zeros((16,), jnp.float32)
    sa = (zero16,) * nb
    na = (zero16,) * nb
    one = jnp.full((16,), 1.0, jnp.float32)
    for c in range(nchunks):
        pltpu.sync_copy(x_hbm.at[pl.ds(row * hw + c * _CH, _CH)], buf)

        def p2_body(i, carry):
            s_l, n_l = list(carry[0]), list(carry[1])
            for u in range(nb):
                v = buf[pl.ds((i * nb + u) * 16, 16)]
                gt = v > t0v
                s_l[u] = s_l[u] + jnp.where(gt, v, zero16)
                n_l[u] = n_l[u] + jnp.where(gt, one, zero16)
            return tuple(s_l), tuple(n_l)

        sa, na = lax.fori_loop(0, _CH // (16 * nb), p2_body, (sa, na))
    s_tot = sa[0]
    n_tot = na[0]
    for p in range(1, nb):
        s_tot = s_tot + sa[p]
        n_tot = n_tot + na[p]
    s_gt = _reduce16(s_tot, redscr, lambda a, b: a + b)
    n_gt = _reduce16(n_tot, redscr, lambda a, b: a + b)
    y = s_gt + t0 * (jnp.float32(_K) - n_gt)
    return y, n_gt


def _make_sc_main(n, hw):
    rows_per_w = n // _NW
    mesh = plsc.VectorSubcoreMesh(core_axis_name="c", subcore_axis_name="s")

    @functools.partial(
        pl.kernel,
        mesh=mesh,
        out_type=[jax.ShapeDtypeStruct((_NW, 16), jnp.float32),
                  jax.ShapeDtypeStruct((_NW, 16), jnp.float32)],
        scratch_types=[pltpu.VMEM((_CH,), jnp.float32),
                       pltpu.VMEM((16,), jnp.float32),
                       pltpu.VMEM((16,), jnp.float32),
                       pltpu.VMEM((16,), jnp.float32)],
    )
    def sc_main(x_hbm, y_hbm, n_hbm, buf, yscr, nscr, redscr):
        wid = lax.axis_index("s") * _NC + lax.axis_index("c")
        lanes = lax.iota(jnp.int32, 16)

        def row_loop(ri, carry):
            y_vec, n_vec = carry
            row = wid * rows_per_w + ri
            y, n_gt = _sc_row_body(x_hbm, buf, redscr, row, hw)
            sel = lanes == ri
            y_vec = jnp.where(sel, jnp.full((16,), y, jnp.float32), y_vec)
            n_vec = jnp.where(sel, jnp.full((16,), n_gt, jnp.float32), n_vec)
            return y_vec, n_vec

        y_vec, n_vec = lax.fori_loop(
            0, rows_per_w, row_loop,
            (jnp.zeros((16,), jnp.float32), jnp.zeros((16,), jnp.float32)))
        yscr[...] = y_vec
        nscr[...] = n_vec
        pltpu.sync_copy(yscr, y_hbm.at[wid])
        pltpu.sync_copy(nscr, n_hbm.at[wid])

    return sc_main


def _oddeven_sort_pairs(nelem):
    pairs = []

    def merge(lo, m, r):
        step = r * 2
        if step < m:
            merge(lo, m, step)
            merge(lo + r, m, step)
            for i in range(lo + r, lo + m - r, step):
                pairs.append((i, i + r))
        else:
            pairs.append((lo, lo + r))

    def sortnet(lo, m):
        if m > 1:
            h = m // 2
            sortnet(lo, h)
            sortnet(lo + h, h)
            merge(lo, m, 1)

    sortnet(0, nelem)
    return pairs


_SORT_PAIRS = _oddeven_sort_pairs(_GRP)


def _cmpx(lst, i, j):
    a, b = lst[i], lst[j]
    lst[i] = jnp.maximum(a, b)
    lst[j] = jnp.minimum(a, b)


def _merge_keep_top16(state, other):
    v = [jnp.maximum(state[k], other[_K - 1 - k]) for k in range(_K)]
    for d in (8, 4, 2, 1):
        for i in range(_K):
            if not i & d:
                _cmpx(v, i, i + d)
    return v


def _merge_equal(a, b):
    d = len(a)
    v = list(a) + list(b)[::-1]
    dist = d
    while dist >= 1:
        for i in range(2 * d):
            if not i & dist:
                _cmpx(v, i, i + dist)
        dist //= 2
    return v


def _fold_sorted(state):
    for axis, shift in _FOLDS:
        rolled = [jnp.roll(s, shift, axis=axis) for s in state]
        if len(state) < _K:
            state = _merge_equal(state, rolled)
        else:
            state = _merge_keep_top16(state, rolled)
    return state


def _rescue_kernel(x_ref, o_ref):
    nchunks = x_ref.shape[1] // 8
    neg = jnp.float32(-jnp.inf)
    init = [jnp.full((8, 128), neg, dtype=jnp.float32) for _ in range(_K)]
    ngroups = nchunks // _GRP

    def insert_body(g, state):
        ch = [x_ref[0, pl.ds((g * _GRP + j) * 8, 8), :] for j in range(_GRP)]
        for (i, j) in _SORT_PAIRS:
            _cmpx(ch, i, j)
        return _merge_keep_top16(state, ch)

    state = jax.lax.fori_loop(0, ngroups, insert_body, init)
    state = _fold_sorted(state)
    total = state[0]
    for k in range(1, _K):
        total = total + state[k]
    o_ref[0] = total


def kernel(x):
    b, c, h, w = x.shape
    n = b * c
    hw = h * w
    assert hw % (1024 * _GRP) == 0
    assert n % _NW == 0 and hw % _CH == 0
    rows = hw // 128

    x1d = x.reshape(n * hw)
    y_t, n_t = _make_sc_main(n, hw)(x1d)
    rows_per_w = n // _NW
    y = y_t[:, :rows_per_w].reshape(n)
    n_gt = n_t[:, :rows_per_w].reshape(n)

    def rescue():
        xr = x.reshape(n, rows, 128)
        out = pl.pallas_call(
            _rescue_kernel,
            grid=(n,),
            in_specs=[pl.BlockSpec((1, rows, 128), lambda i: (i, 0, 0))],
            out_specs=pl.BlockSpec((1, 8, 128), lambda i: (i, 0, 0)),
            out_shape=jax.ShapeDtypeStruct((n, 8, 128), jnp.float32),
        )(xr)
        return out[:, 0, 0]

    y = jax.lax.cond(jnp.any(n_gt > jnp.float32(_K - 1)), rescue, lambda: y)
    return y.reshape(b, c)
